# Initial kernel scaffold; baseline (speedup 1.0000x reference)
import jax
import jax.numpy as jnp
from jax.experimental import pallas as pl


def kernel(z_mp_i1, z_mp_i2, W1, b1, W2, b2):
    raise NotImplementedError("write your pallas kernel here")


# speedup vs baseline: 1.4717x; 1.4717x over previous
import jax
import jax.numpy as jnp
from jax.experimental import pallas as pl
from jax.experimental.pallas import tpu as pltpu

N = 4096
D = 64
TAU = 0.5
BLK = 512
NB = N // BLK


def _ssl_body(z1_ref, z2_ref, w1_ref, b1_ref, w2_ref, b2_ref,
              out_ref, z2pn_ref, acc_ref):
    i = pl.program_id(0)

    w1t = w1_ref[...].T
    w2t = w2_ref[...].T
    b1 = b1_ref[...]
    b2 = b2_ref[...]

    @pl.when(i == 0)
    def _init():
        z2 = z2_ref[...]
        h2 = jnp.maximum(
            jax.lax.dot(z2, w1t, preferred_element_type=jnp.float32) + b1, 0.0)
        z2p = jax.lax.dot(h2, w2t, preferred_element_type=jnp.float32) + b2
        n2 = jnp.sqrt(jnp.sum(z2p * z2p, axis=1, keepdims=True))
        z2pn_ref[...] = z2p / n2
        acc_ref[0] = 0.0
        acc_ref[1] = 0.0

    z1 = z1_ref[...]
    h1 = jnp.maximum(
        jax.lax.dot(z1, w1t, preferred_element_type=jnp.float32) + b1, 0.0)
    z1p = jax.lax.dot(h1, w2t, preferred_element_type=jnp.float32) + b2
    n1 = jnp.sqrt(jnp.sum(z1p * z1p, axis=1, keepdims=True))
    z1pn = z1p / (n1 * TAU)

    s = jax.lax.dot_general(z1pn, z2pn_ref[...],
                            (((1,), (1,)), ((), ())),
                            preferred_element_type=jnp.float32)
    rowsum = jnp.sum(jnp.exp(s), axis=1, keepdims=True)
    acc_ref[0] += jnp.sum(s)
    acc_ref[1] += jnp.sum(jnp.log(rowsum + 1e-8))

    @pl.when(i == NB - 1)
    def _fin():
        out_ref[0] = -acc_ref[0] / (N * N) + acc_ref[1] / N


@jax.jit
def kernel(z_mp_i1, z_mp_i2, W1, b1, W2, b2):
    b1r = b1.reshape(1, D)
    b2r = b2.reshape(1, D)
    out = pl.pallas_call(
        _ssl_body,
        grid=(NB,),
        in_specs=[
            pl.BlockSpec((BLK, D), lambda i: (i, 0)),
            pl.BlockSpec((N, D), lambda i: (0, 0)),
            pl.BlockSpec((D, D), lambda i: (0, 0)),
            pl.BlockSpec((1, D), lambda i: (0, 0)),
            pl.BlockSpec((D, D), lambda i: (0, 0)),
            pl.BlockSpec((1, D), lambda i: (0, 0)),
        ],
        out_specs=pl.BlockSpec(memory_space=pltpu.SMEM),
        out_shape=jax.ShapeDtypeStruct((1,), jnp.float32),
        scratch_shapes=[
            pltpu.VMEM((N, D), jnp.float32),
            pltpu.SMEM((2,), jnp.float32),
        ],
    )(z_mp_i1, z_mp_i2, W1, b1r, W2, b2r)
    return out[0]


# speedup vs baseline: 2.2813x; 1.5501x over previous
import jax
import jax.numpy as jnp
from jax.experimental import pallas as pl
from jax.experimental.pallas import tpu as pltpu

N = 4096
D = 64
TAU = 0.5
BLK = 512
NB = N // BLK
LOG2E = 1.4426950408889634
SCALE = LOG2E / TAU


def _ssl_body(z1_ref, z2_ref, w1_ref, b1_ref, w2_ref, b2_ref,
              out_ref, z2pn_ref, z2sum_ref, z1sum_ref, acc_ref):
    i = pl.program_id(0)

    w1t = w1_ref[...].T
    w2t = w2_ref[...].T
    b1 = b1_ref[...]
    b2 = b2_ref[...]

    @pl.when(i == 0)
    def _init():
        z2 = z2_ref[...]
        h2 = jnp.maximum(
            jax.lax.dot(z2, w1t, preferred_element_type=jnp.float32) + b1, 0.0)
        z2p = jax.lax.dot(h2, w2t, preferred_element_type=jnp.float32) + b2
        n2 = jnp.sqrt(jnp.sum(z2p * z2p, axis=1, keepdims=True))
        z2pn = z2p / n2
        z2pn_ref[...] = z2pn.astype(jnp.bfloat16)
        z2sum_ref[...] = jnp.sum(z2pn, axis=0, keepdims=True)
        z1sum_ref[...] = jnp.zeros_like(z1sum_ref)
        acc_ref[0] = 0.0

    z1 = z1_ref[...]
    h1 = jnp.maximum(
        jax.lax.dot(z1, w1t, preferred_element_type=jnp.float32) + b1, 0.0)
    z1p = jax.lax.dot(h1, w2t, preferred_element_type=jnp.float32) + b2
    n1 = jnp.sqrt(jnp.sum(z1p * z1p, axis=1, keepdims=True))
    z1pn = z1p * (SCALE / n1)
    z1sum_ref[...] += jnp.sum(z1pn, axis=0, keepdims=True)

    s2 = jax.lax.dot_general(z1pn.astype(jnp.bfloat16), z2pn_ref[...],
                             (((1,), (1,)), ((), ())),
                             preferred_element_type=jnp.float32)
    rowsum = jnp.sum(jnp.exp2(s2), axis=1, keepdims=True)
    acc_ref[0] += jnp.sum(jnp.log(rowsum + 1e-8))

    @pl.when(i == NB - 1)
    def _fin():
        total_s = jnp.sum(z1sum_ref[...] * z2sum_ref[...]) * (1.0 / LOG2E)
        out_ref[0] = -total_s / (N * N) + acc_ref[0] / N


@jax.jit
def kernel(z_mp_i1, z_mp_i2, W1, b1, W2, b2):
    b1r = b1.reshape(1, D)
    b2r = b2.reshape(1, D)
    out = pl.pallas_call(
        _ssl_body,
        grid=(NB,),
        in_specs=[
            pl.BlockSpec((BLK, D), lambda i: (i, 0)),
            pl.BlockSpec((N, D), lambda i: (0, 0)),
            pl.BlockSpec((D, D), lambda i: (0, 0)),
            pl.BlockSpec((1, D), lambda i: (0, 0)),
            pl.BlockSpec((D, D), lambda i: (0, 0)),
            pl.BlockSpec((1, D), lambda i: (0, 0)),
        ],
        out_specs=pl.BlockSpec(memory_space=pltpu.SMEM),
        out_shape=jax.ShapeDtypeStruct((1,), jnp.float32),
        scratch_shapes=[
            pltpu.VMEM((N, D), jnp.bfloat16),
            pltpu.VMEM((1, D), jnp.float32),
            pltpu.VMEM((1, D), jnp.float32),
            pltpu.SMEM((2,), jnp.float32),
        ],
    )(z_mp_i1, z_mp_i2, W1, b1r, W2, b2r)
    return out[0]
